# baseline (speedup 1.0000x reference)
import functools

import jax
import jax.numpy as jnp
from jax import lax
from jax.experimental import pallas as pl
from jax.experimental.pallas import tpu as pltpu

N = 10000
E = 320000
IN_DIM = 128
HID = 1024
NUM_GRAPHS = 64
NCHUNK = HID // 128
MB = 1000
NMB = N // MB


def _mm1_body(x_ref, aggp_ref, degp_ref, wl_ref, bl_ref, wr_ref, out_ref):
    deg = degp_ref[0, :, 0] + degp_ref[1, :, 0]
    invd = 1.0 / jnp.maximum(deg, 1.0)
    agg = aggp_ref[0] + aggp_ref[1]
    mean = agg * invd[:, None]
    h = jnp.dot(mean, wl_ref[...], preferred_element_type=jnp.float32)
    h = h + bl_ref[0, 0][None, :]
    h = h + jnp.dot(x_ref[...], wr_ref[...], preferred_element_type=jnp.float32)
    out_ref[0] = jnp.tanh(h)


def _mm1(x, aggp, degp, Wl1, bl1, Wr1):
    return pl.pallas_call(
        _mm1_body,
        grid=(NMB, NCHUNK),
        in_specs=[
            pl.BlockSpec((MB, IN_DIM), lambda i, c: (i, 0)),
            pl.BlockSpec((2, MB, IN_DIM), lambda i, c: (0, i, 0)),
            pl.BlockSpec((2, MB, 16), lambda i, c: (0, i, 0)),
            pl.BlockSpec((IN_DIM, 128), lambda i, c: (0, c)),
            pl.BlockSpec((1, 1, 128), lambda i, c: (c, 0, 0)),
            pl.BlockSpec((IN_DIM, 128), lambda i, c: (0, c)),
        ],
        out_specs=pl.BlockSpec((1, MB, 128), lambda i, c: (c, i, 0)),
        out_shape=jax.ShapeDtypeStruct((NCHUNK, N, 128), jnp.float32),
    )(x, aggp, degp, Wl1, bl1.reshape(NCHUNK, 1, 128), Wr1)


def _mm2_body(h1_ref, agg2_ref, degp_ref, wl_ref, bl_ref, wr_ref, b_ref,
              out_ref):
    deg = degp_ref[0, :, 0] + degp_ref[1, :, 0]
    invd = 1.0 / jnp.maximum(deg, 1.0)
    s1 = jnp.zeros((MB, HID), jnp.float32)
    s2 = jnp.zeros((MB, HID), jnp.float32)
    for c in range(NCHUNK):
        wl = wl_ref[pl.ds(c * 128, 128), :]
        wr = wr_ref[pl.ds(c * 128, 128), :]
        s1 = s1 + jnp.dot(agg2_ref[c], wl, preferred_element_type=jnp.float32)
        s2 = s2 + jnp.dot(h1_ref[c], wr, preferred_element_type=jnp.float32)
    h2 = jnp.tanh(s1 * invd[:, None] + bl_ref[0][None, :] + s2)
    b = b_ref[0, 0, :]
    onehot = (b[None, :] == lax.broadcasted_iota(jnp.int32, (NUM_GRAPHS, MB), 0)
              ).astype(jnp.float32)
    gp = jnp.dot(onehot, h2, preferred_element_type=jnp.float32)

    @pl.when(pl.program_id(0) == 0)
    def _():
        out_ref[...] = jnp.zeros_like(out_ref)

    out_ref[...] += gp


def _mm2(h1c, agg2c, degp, Wl2, bl2, Wr2, batch):
    return pl.pallas_call(
        _mm2_body,
        grid=(NMB,),
        in_specs=[
            pl.BlockSpec((NCHUNK, MB, 128), lambda i: (0, i, 0)),
            pl.BlockSpec((NCHUNK, MB, 128), lambda i: (0, i, 0)),
            pl.BlockSpec((2, MB, 16), lambda i: (0, i, 0)),
            pl.BlockSpec((HID, HID), lambda i: (0, 0)),
            pl.BlockSpec((1, HID), lambda i: (0, 0)),
            pl.BlockSpec((HID, HID), lambda i: (0, 0)),
            pl.BlockSpec((1, 1, MB), lambda i: (i, 0, 0)),
        ],
        out_specs=pl.BlockSpec((NUM_GRAPHS, HID), lambda i: (0, 0)),
        out_shape=jax.ShapeDtypeStruct((NUM_GRAPHS, HID), jnp.float32),
    )(h1c, agg2c, degp, Wl2, bl2.reshape(1, HID), Wr2,
      batch.reshape(NMB, 1, MB))


def _mm3_body(g_ref, wf1_ref, bf1_ref, wf2_ref, bf2_ref, out_ref):
    g1 = jnp.dot(g_ref[...], wf1_ref[...], preferred_element_type=jnp.float32)
    g1 = g1 + bf1_ref[0][None, :]
    g1 = jnp.where(g1 >= 0, g1, 0.2 * g1)
    o = jnp.dot(g1, wf2_ref[...], preferred_element_type=jnp.float32)
    out_ref[...] = o + bf2_ref[0][None, :]


def _mm3(g, Wf1, bf1, Wf2, bf2):
    return pl.pallas_call(
        _mm3_body,
        out_shape=jax.ShapeDtypeStruct((NUM_GRAPHS, 1), jnp.float32),
    )(g, Wf1, bf1.reshape(1, 128), Wf2, bf2.reshape(1, 1))


def _aggregate(table, src, dst, nrows):
    msgs = jnp.take(table, src, axis=0)
    agg = jax.ops.segment_sum(msgs, dst, num_segments=nrows)
    return agg


def kernel(x, edge_index, batch, Wl1, bl1, Wr1, Wl2, bl2, Wr2, Wf1, bf1,
           Wf2, bf2):
    src = edge_index[0]
    dst = edge_index[1]

    agg1 = _aggregate(x, src, dst, N)
    aggp = jnp.stack([agg1, jnp.zeros_like(agg1)])
    deg = jax.ops.segment_sum(jnp.ones((E,), jnp.float32), dst, num_segments=N)
    degp = jnp.zeros((2, N, 16), jnp.float32).at[0, :, 0].set(deg)

    h1c = _mm1(x, aggp, degp, Wl1, bl1, Wr1)

    h1 = h1c.transpose(1, 0, 2).reshape(N, HID)
    agg2 = _aggregate(h1, src, dst, N)
    agg2c = agg2.reshape(N, NCHUNK, 128).transpose(1, 0, 2)

    g = _mm2(h1c, agg2c, degp, Wl2, bl2, Wr2, batch)
    return _mm3(g, Wf1, bf1, Wf2, bf2)


# speedup vs baseline: 3.2711x; 3.2711x over previous
import functools

import jax
import jax.numpy as jnp
from jax import lax
from jax.experimental import pallas as pl
from jax.experimental.pallas import tpu as pltpu
from jax.experimental.pallas import tpu_sc as plsc

N = 10000
E = 320000
IN_DIM = 128
HID = 1024
NUM_GRAPHS = 64
NCHUNK = HID // 128
MB = 1000
NMB = N // MB


def _mm1_body(x_ref, aggp_ref, degp_ref, wl_ref, bl_ref, wr_ref,
              out_ref, dinv_ref):
    deg = degp_ref[0, :, 0] + degp_ref[1, :, 0]
    invd = 1.0 / jnp.maximum(deg, 1.0)
    agg = aggp_ref[0] + aggp_ref[1]
    mean = agg * invd[:, None]
    h = jnp.dot(mean, wl_ref[...], preferred_element_type=jnp.float32)
    h = h + bl_ref[0, 0][None, :]
    h = h + jnp.dot(x_ref[...], wr_ref[...], preferred_element_type=jnp.float32)
    out_ref[0] = jnp.tanh(h)
    dinv_ref[...] = jnp.broadcast_to(invd[:, None], (MB, 16))


def _mm1(x, aggp, degp, Wl1, bl1, Wr1):
    return pl.pallas_call(
        _mm1_body,
        grid=(NMB, NCHUNK),
        in_specs=[
            pl.BlockSpec((MB, IN_DIM), lambda i, c: (i, 0)),
            pl.BlockSpec((2, MB, 128), lambda i, c: (0, i, 0)),
            pl.BlockSpec((2, MB, 128), lambda i, c: (0, i, 0)),
            pl.BlockSpec((IN_DIM, 128), lambda i, c: (0, c)),
            pl.BlockSpec((1, 1, 128), lambda i, c: (c, 0, 0)),
            pl.BlockSpec((IN_DIM, 128), lambda i, c: (0, c)),
        ],
        out_specs=[pl.BlockSpec((1, MB, 128), lambda i, c: (c, i, 0)),
                   pl.BlockSpec((MB, 16), lambda i, c: (i, 0))],
        out_shape=[jax.ShapeDtypeStruct((NCHUNK, N, 128), jnp.float32),
                   jax.ShapeDtypeStruct((N, 16), jnp.float32)],
    )(x, aggp, degp, Wl1, bl1.reshape(NCHUNK, 1, 128), Wr1)


def _mm2_body(h1_ref, agg2_ref, dinv_ref, wl_ref, bl_ref, wr_ref, b_ref,
              out_ref):
    invd = dinv_ref[:, 0]
    s1 = jnp.zeros((MB, HID), jnp.float32)
    s2 = jnp.zeros((MB, HID), jnp.float32)
    for c in range(NCHUNK):
        wl = wl_ref[pl.ds(c * 128, 128), :]
        wr = wr_ref[pl.ds(c * 128, 128), :]
        s1 = s1 + jnp.dot(agg2_ref[c], wl, preferred_element_type=jnp.float32)
        s2 = s2 + jnp.dot(h1_ref[c], wr, preferred_element_type=jnp.float32)
    h2 = jnp.tanh(s1 * invd[:, None] + bl_ref[0][None, :] + s2)
    b = b_ref[0, 0, :]
    onehot = (b[None, :] == lax.broadcasted_iota(jnp.int32, (NUM_GRAPHS, MB), 0)
              ).astype(jnp.float32)
    gp = jnp.dot(onehot, h2, preferred_element_type=jnp.float32)

    @pl.when(pl.program_id(0) == 0)
    def _():
        out_ref[...] = jnp.zeros_like(out_ref)

    out_ref[...] += gp


def _mm2(h1c, agg2c, dinv, Wl2, bl2, Wr2, batch):
    return pl.pallas_call(
        _mm2_body,
        grid=(NMB,),
        in_specs=[
            pl.BlockSpec((NCHUNK, MB, 128), lambda i: (0, i, 0)),
            pl.BlockSpec((NCHUNK, MB, 128), lambda i: (0, i, 0)),
            pl.BlockSpec((MB, 16), lambda i: (i, 0)),
            pl.BlockSpec((HID, HID), lambda i: (0, 0)),
            pl.BlockSpec((1, HID), lambda i: (0, 0)),
            pl.BlockSpec((HID, HID), lambda i: (0, 0)),
            pl.BlockSpec((1, 1, MB), lambda i: (i, 0, 0)),
        ],
        out_specs=pl.BlockSpec((NUM_GRAPHS, HID), lambda i: (0, 0)),
        out_shape=jax.ShapeDtypeStruct((NUM_GRAPHS, HID), jnp.float32),
    )(h1c, agg2c, dinv, Wl2, bl2.reshape(1, HID), Wr2,
      batch.reshape(NMB, 1, MB))


def _mm3_body(g_ref, wf1_ref, bf1_ref, wf2_ref, bf2_ref, out_ref):
    g1 = jnp.dot(g_ref[...], wf1_ref[...], preferred_element_type=jnp.float32)
    g1 = g1 + bf1_ref[0][None, :]
    g1 = jnp.where(g1 >= 0, g1, 0.2 * g1)
    o = jnp.dot(g1, wf2_ref[...], preferred_element_type=jnp.float32)
    out_ref[...] = o + bf2_ref[0][None, :]


def _mm3(g, Wf1, bf1, Wf2, bf2):
    return pl.pallas_call(
        _mm3_body,
        out_shape=jax.ShapeDtypeStruct((NUM_GRAPHS, 1), jnp.float32),
    )(g, Wf1, bf1.reshape(1, 128), Wf2, bf2.reshape(1, 1))


NCORE = 2
NTILE = 16
BATCH = 128
NB = E // BATCH
PIECE = 128
NPIECE = (N + PIECE - 1) // PIECE

def _sc_mesh():
    return plsc.VectorSubcoreMesh(
        core_axis_name="c", subcore_axis_name="s",
        num_cores=NCORE, num_subcores=NTILE)


def _fill_iota(ibuf, start, n):
    for j in range(n // 16):
        ibuf[pl.ds(j * 16, 16)] = (
            lax.iota(jnp.int32, 16) + (start + j * 16))


def _piece_start(t):
    return jnp.minimum(t * PIECE, N - PIECE)


def _sc1_body(x_hbm, src_hbm, dst_hbm, z128_hbm, ones_hbm,
              outagg, outdeg, acc, sbuf, dbuf, rows, ibuf, sem):
    cid = lax.axis_index("c")
    sid = lax.axis_index("s")

    w = sid * NCORE + cid

    def zero_acc(buf):
        def zbody(k, carry):
            t = sid + k * NTILE

            @pl.when(t < NPIECE)
            def _():
                _fill_iota(ibuf, _piece_start(t), PIECE)
                pltpu.sync_copy(buf, acc.at[ibuf])

            return carry

        lax.fori_loop(0, NPIECE // NTILE + 1, zbody, 0)

    def writeout(dst_ref):
        def wrbody(k, carry):
            t = sid + k * NTILE

            @pl.when(t < NPIECE)
            def _():
                p = _piece_start(t)
                _fill_iota(ibuf, p, PIECE)
                pltpu.sync_copy(acc.at[ibuf], rows)
                pltpu.sync_copy(rows, dst_ref.at[cid, pl.ds(p, PIECE), :])

            return carry

        lax.fori_loop(0, NPIECE // NTILE + 1, wrbody, 0)

    pltpu.sync_copy(z128_hbm, rows)
    zero_acc(rows)
    plsc.subcore_barrier()

    def body(i, carry):
        b = w + i * (NCORE * NTILE)

        @pl.when(b < NB)
        def _():
            base = b * BATCH
            pltpu.sync_copy(src_hbm.at[pl.ds(base, BATCH)], sbuf)
            pltpu.sync_copy(dst_hbm.at[pl.ds(base, BATCH)], dbuf)
            pltpu.async_copy(x_hbm.at[sbuf], rows, sem).wait()
            pltpu.sync_copy(rows, acc.at[dbuf], add=True)

        return carry

    lax.fori_loop(0, NB // (NCORE * NTILE) + 1, body, 0)
    plsc.subcore_barrier()
    writeout(outagg)
    plsc.subcore_barrier()

    pltpu.sync_copy(z128_hbm, rows)
    zero_acc(rows)
    pltpu.sync_copy(ones_hbm, rows)
    plsc.subcore_barrier()

    def dbody(i, carry):
        b = w + i * (NCORE * NTILE)

        @pl.when(b < NB)
        def _():
            base = b * BATCH
            pltpu.sync_copy(dst_hbm.at[pl.ds(base, BATCH)], dbuf)
            pltpu.sync_copy(rows, acc.at[dbuf], add=True)

        return carry

    lax.fori_loop(0, NB // (NCORE * NTILE) + 1, dbody, 0)
    plsc.subcore_barrier()
    writeout(outdeg)


def _sc_agg1(x, src, dst):
    z128 = jnp.zeros((PIECE, 128), jnp.float32)
    ones = jnp.ones((BATCH, 128), jnp.float32)
    return pl.kernel(
        _sc1_body,
        out_type=[jax.ShapeDtypeStruct((NCORE, N, 128), jnp.float32),
                  jax.ShapeDtypeStruct((NCORE, N, 128), jnp.float32)],
        mesh=_sc_mesh(),
        scratch_types=[
            pltpu.VMEM_SHARED((N, 128), jnp.float32),
            pltpu.VMEM((BATCH,), jnp.int32),
            pltpu.VMEM((BATCH,), jnp.int32),
            pltpu.VMEM((BATCH, 128), jnp.float32),
            pltpu.VMEM((PIECE,), jnp.int32),
            pltpu.SemaphoreType.DMA,
        ],
    )(x, src, dst, z128, ones)


def _sc2_body(h1_hbm, src_hbm, dst_hbm, z128_hbm, out, acc,
              sbuf, dbuf, gbuf, rows, ibuf, sem):
    cid = lax.axis_index("c")
    sid = lax.axis_index("s")

    for q in range(NCHUNK // NCORE):
        c = cid * (NCHUNK // NCORE) + q

        pltpu.sync_copy(z128_hbm, rows)

        def zbody(k, carry):
            t = sid + k * NTILE

            @pl.when(t < NPIECE)
            def _():
                _fill_iota(ibuf, _piece_start(t), PIECE)
                pltpu.sync_copy(rows, acc.at[ibuf])

            return carry

        lax.fori_loop(0, NPIECE // NTILE + 1, zbody, 0)
        plsc.subcore_barrier()

        def body(i, carry):
            b = sid + i * NTILE

            @pl.when(b < NB)
            def _():
                base = b * BATCH
                pltpu.sync_copy(src_hbm.at[pl.ds(base, BATCH)], sbuf)
                pltpu.sync_copy(dst_hbm.at[pl.ds(base, BATCH)], dbuf)
                for j in range(BATCH // 16):
                    gbuf[pl.ds(j * 16, 16)] = (
                        sbuf[pl.ds(j * 16, 16)] + c * N)
                pltpu.async_copy(h1_hbm.at[gbuf], rows, sem).wait()
                pltpu.sync_copy(rows, acc.at[dbuf], add=True)

            return carry

        lax.fori_loop(0, NB // NTILE + 1, body, 0)
        plsc.subcore_barrier()

        def wrbody(k, carry):
            t = sid + k * NTILE

            @pl.when(t < NPIECE)
            def _():
                p = _piece_start(t)
                _fill_iota(ibuf, p, PIECE)
                pltpu.sync_copy(acc.at[ibuf], rows)
                pltpu.sync_copy(rows, out.at[pl.ds(c * N + p, PIECE)])

            return carry

        lax.fori_loop(0, NPIECE // NTILE + 1, wrbody, 0)
        plsc.subcore_barrier()


def _sc_agg2(h1flat, src, dst):
    z128 = jnp.zeros((PIECE, 128), jnp.float32)
    return pl.kernel(
        _sc2_body,
        out_type=jax.ShapeDtypeStruct((NCHUNK * N, 128), jnp.float32),
        mesh=_sc_mesh(),
        scratch_types=[
            pltpu.VMEM_SHARED((N, 128), jnp.float32),
            pltpu.VMEM((BATCH,), jnp.int32),
            pltpu.VMEM((BATCH,), jnp.int32),
            pltpu.VMEM((BATCH,), jnp.int32),
            pltpu.VMEM((BATCH, 128), jnp.float32),
            pltpu.VMEM((PIECE,), jnp.int32),
            pltpu.SemaphoreType.DMA,
        ],
    )(h1flat, src, dst, z128)


def kernel(x, edge_index, batch, Wl1, bl1, Wr1, Wl2, bl2, Wr2, Wf1, bf1,
           Wf2, bf2):
    src = edge_index[0]
    dst = edge_index[1]

    aggp, degp = _sc_agg1(x, src, dst)

    h1c, dinv = _mm1(x, aggp, degp, Wl1, bl1, Wr1)

    agg2c = _sc_agg2(h1c.reshape(NCHUNK * N, 128), src, dst
                     ).reshape(NCHUNK, N, 128)

    g = _mm2(h1c, agg2c, dinv, Wl2, bl2, Wr2, batch)
    return _mm3(g, Wf1, bf1, Wf2, bf2)


# speedup vs baseline: 4.6562x; 1.4235x over previous
import functools

import jax
import jax.numpy as jnp
from jax import lax
from jax.experimental import pallas as pl
from jax.experimental.pallas import tpu as pltpu
from jax.experimental.pallas import tpu_sc as plsc

N = 10000
E = 320000
IN_DIM = 128
HID = 1024
NUM_GRAPHS = 64
NCHUNK = HID // 128
MB = 1000
NMB = N // MB


def _mm1_body(x_ref, aggp_ref, degp_ref, wl_ref, bl_ref, wr_ref,
              out_ref, dinv_ref):
    deg = degp_ref[0, :, 0] + degp_ref[1, :, 0]
    invd = 1.0 / jnp.maximum(deg, 1.0)
    agg = aggp_ref[0] + aggp_ref[1]
    mean = agg * invd[:, None]
    h = jnp.dot(mean, wl_ref[...], preferred_element_type=jnp.float32)
    h = h + bl_ref[0, 0][None, :]
    h = h + jnp.dot(x_ref[...], wr_ref[...], preferred_element_type=jnp.float32)
    out_ref[0] = jnp.tanh(h)
    dinv_ref[...] = jnp.broadcast_to(invd[:, None], (MB, 16))


def _mm1(x, aggp, degp, Wl1, bl1, Wr1):
    return pl.pallas_call(
        _mm1_body,
        grid=(NMB, NCHUNK),
        in_specs=[
            pl.BlockSpec((MB, IN_DIM), lambda i, c: (i, 0)),
            pl.BlockSpec((2, MB, 128), lambda i, c: (0, i, 0)),
            pl.BlockSpec((2, MB, 128), lambda i, c: (0, i, 0)),
            pl.BlockSpec((IN_DIM, 128), lambda i, c: (0, c)),
            pl.BlockSpec((1, 1, 128), lambda i, c: (c, 0, 0)),
            pl.BlockSpec((IN_DIM, 128), lambda i, c: (0, c)),
        ],
        out_specs=[pl.BlockSpec((1, MB, 128), lambda i, c: (c, i, 0)),
                   pl.BlockSpec((MB, 16), lambda i, c: (i, 0))],
        out_shape=[jax.ShapeDtypeStruct((NCHUNK, N, 128), jnp.float32),
                   jax.ShapeDtypeStruct((N, 16), jnp.float32)],
    )(x, aggp, degp, Wl1, bl1.reshape(NCHUNK, 1, 128), Wr1)


def _mm2_body(h1_ref, agg2_ref, dinv_ref, wl_ref, bl_ref, wr_ref, b_ref,
              out_ref):
    invd = dinv_ref[:, 0]
    s1 = jnp.zeros((MB, HID), jnp.float32)
    s2 = jnp.zeros((MB, HID), jnp.float32)
    for c in range(NCHUNK):
        wl = wl_ref[pl.ds(c * 128, 128), :]
        wr = wr_ref[pl.ds(c * 128, 128), :]
        s1 = s1 + jnp.dot(agg2_ref[c], wl, preferred_element_type=jnp.float32)
        s2 = s2 + jnp.dot(h1_ref[c], wr, preferred_element_type=jnp.float32)
    h2 = jnp.tanh(s1 * invd[:, None] + bl_ref[0][None, :] + s2)
    b = b_ref[0, 0, :]
    onehot = (b[None, :] == lax.broadcasted_iota(jnp.int32, (NUM_GRAPHS, MB), 0)
              ).astype(jnp.float32)
    gp = jnp.dot(onehot, h2, preferred_element_type=jnp.float32)

    @pl.when(pl.program_id(0) == 0)
    def _():
        out_ref[...] = jnp.zeros_like(out_ref)

    out_ref[...] += gp


def _mm2(h1c, agg2c, dinv, Wl2, bl2, Wr2, batch):
    return pl.pallas_call(
        _mm2_body,
        grid=(NMB,),
        in_specs=[
            pl.BlockSpec((NCHUNK, MB, 128), lambda i: (0, i, 0)),
            pl.BlockSpec((NCHUNK, MB, 128), lambda i: (0, i, 0)),
            pl.BlockSpec((MB, 16), lambda i: (i, 0)),
            pl.BlockSpec((HID, HID), lambda i: (0, 0)),
            pl.BlockSpec((1, HID), lambda i: (0, 0)),
            pl.BlockSpec((HID, HID), lambda i: (0, 0)),
            pl.BlockSpec((1, 1, MB), lambda i: (i, 0, 0)),
        ],
        out_specs=pl.BlockSpec((NUM_GRAPHS, HID), lambda i: (0, 0)),
        out_shape=jax.ShapeDtypeStruct((NUM_GRAPHS, HID), jnp.float32),
    )(h1c, agg2c, dinv, Wl2, bl2.reshape(1, HID), Wr2,
      batch.reshape(NMB, 1, MB))


def _mm3_body(g_ref, wf1_ref, bf1_ref, wf2_ref, bf2_ref, out_ref):
    g1 = jnp.dot(g_ref[...], wf1_ref[...], preferred_element_type=jnp.float32)
    g1 = g1 + bf1_ref[0][None, :]
    g1 = jnp.where(g1 >= 0, g1, 0.2 * g1)
    o = jnp.dot(g1, wf2_ref[...], preferred_element_type=jnp.float32)
    out_ref[...] = o + bf2_ref[0][None, :]


def _mm3(g, Wf1, bf1, Wf2, bf2):
    return pl.pallas_call(
        _mm3_body,
        out_shape=jax.ShapeDtypeStruct((NUM_GRAPHS, 1), jnp.float32),
    )(g, Wf1, bf1.reshape(1, 128), Wf2, bf2.reshape(1, 1))


NCORE = 2
NTILE = 16
BATCH = 128
NB = E // BATCH
PIECE = 128
NPIECE = (N + PIECE - 1) // PIECE

def _sc_mesh():
    return plsc.VectorSubcoreMesh(
        core_axis_name="c", subcore_axis_name="s",
        num_cores=NCORE, num_subcores=NTILE)


def _fill_iota(ibuf, start, n):
    for j in range(n // 16):
        ibuf[pl.ds(j * 16, 16)] = (
            lax.iota(jnp.int32, 16) + (start + j * 16))


def _piece_start(t):
    return jnp.minimum(t * PIECE, N - PIECE)


def _sc1_body(x_hbm, src_hbm, dst_hbm, z128_hbm, ones_hbm,
              outagg, outdeg, acc, sbuf, dbuf, rows, ibuf, sem):
    cid = lax.axis_index("c")
    sid = lax.axis_index("s")

    w = sid * NCORE + cid

    def zero_acc(buf):
        def zbody(k, carry):
            t = sid + k * NTILE

            @pl.when(t < NPIECE)
            def _():
                _fill_iota(ibuf, _piece_start(t), PIECE)
                pltpu.sync_copy(buf, acc.at[ibuf])

            return carry

        lax.fori_loop(0, NPIECE // NTILE + 1, zbody, 0)

    def writeout(dst_ref):
        def wrbody(k, carry):
            t = sid + k * NTILE

            @pl.when(t < NPIECE)
            def _():
                p = _piece_start(t)
                _fill_iota(ibuf, p, PIECE)
                pltpu.sync_copy(acc.at[ibuf], rows)
                pltpu.sync_copy(rows, dst_ref.at[cid, pl.ds(p, PIECE), :])

            return carry

        lax.fori_loop(0, NPIECE // NTILE + 1, wrbody, 0)

    pltpu.sync_copy(z128_hbm, rows)
    zero_acc(rows)
    plsc.subcore_barrier()

    def body(i, carry):
        b = w + i * (NCORE * NTILE)

        @pl.when(b < NB)
        def _():
            base = b * BATCH
            pltpu.sync_copy(src_hbm.at[pl.ds(base, BATCH)], sbuf)
            pltpu.sync_copy(dst_hbm.at[pl.ds(base, BATCH)], dbuf)
            pltpu.async_copy(x_hbm.at[sbuf], rows, sem).wait()
            pltpu.sync_copy(rows, acc.at[dbuf], add=True)

        return carry

    lax.fori_loop(0, NB // (NCORE * NTILE) + 1, body, 0)
    plsc.subcore_barrier()
    writeout(outagg)
    plsc.subcore_barrier()

    pltpu.sync_copy(z128_hbm, rows)
    zero_acc(rows)
    pltpu.sync_copy(ones_hbm, rows)
    plsc.subcore_barrier()

    def dbody(i, carry):
        b = w + i * (NCORE * NTILE)

        @pl.when(b < NB)
        def _():
            base = b * BATCH
            pltpu.sync_copy(dst_hbm.at[pl.ds(base, BATCH)], dbuf)
            pltpu.sync_copy(rows, acc.at[dbuf], add=True)

        return carry

    lax.fori_loop(0, NB // (NCORE * NTILE) + 1, dbody, 0)
    plsc.subcore_barrier()
    writeout(outdeg)


def _sc_agg1(x, src, dst):
    z128 = jnp.zeros((PIECE, 128), jnp.float32)
    ones = jnp.ones((BATCH, 128), jnp.float32)
    return pl.kernel(
        _sc1_body,
        out_type=[jax.ShapeDtypeStruct((NCORE, N, 128), jnp.float32),
                  jax.ShapeDtypeStruct((NCORE, N, 128), jnp.float32)],
        mesh=_sc_mesh(),
        scratch_types=[
            pltpu.VMEM_SHARED((N, 128), jnp.float32),
            pltpu.VMEM((BATCH,), jnp.int32),
            pltpu.VMEM((BATCH,), jnp.int32),
            pltpu.VMEM((BATCH, 128), jnp.float32),
            pltpu.VMEM((PIECE,), jnp.int32),
            pltpu.SemaphoreType.DMA,
        ],
    )(x, src, dst, z128, ones)


GRP = 4


def _sc2_body(h1_hbm, src_hbm, dst_hbm, z128_hbm, out, acc,
              sbuf, dbuf, gbuf, rowsA, rowsB, ibuf, semA, semB):
    cid = lax.axis_index("c")
    sid = lax.axis_index("s")

    rows = [rowsA, rowsB]
    sems = [semA, semB]
    NG = NB // GRP

    for q in range(NCHUNK // NCORE):
        c = cid * (NCHUNK // NCORE) + q

        pltpu.sync_copy(z128_hbm, rowsA)

        def zbody(k, carry):
            t = sid + k * NTILE

            @pl.when(t < NPIECE)
            def _():
                _fill_iota(ibuf, _piece_start(t), PIECE)
                pltpu.sync_copy(rowsA, acc.at[ibuf])

            return carry

        lax.fori_loop(0, NPIECE // NTILE + 1, zbody, 0)
        plsc.subcore_barrier()

        def body(i, carry):
            g = sid + i * NTILE

            @pl.when(g < NG)
            def _():
                pltpu.sync_copy(src_hbm.at[g], sbuf)
                pltpu.sync_copy(dst_hbm.at[g], dbuf)
                for j in range(GRP):
                    for k in range(BATCH // 16):
                        gbuf[j, pl.ds(k * 16, 16)] = (
                            sbuf[j, pl.ds(k * 16, 16)] + c * N)
                cps = [None, None]
                cps[0] = pltpu.async_copy(
                    h1_hbm.at[gbuf.at[0]], rows[0], sems[0])
                cps[1] = pltpu.async_copy(
                    h1_hbm.at[gbuf.at[1]], rows[1], sems[1])
                for j in range(GRP):
                    cps[j % 2].wait()
                    pltpu.sync_copy(rows[j % 2], acc.at[dbuf.at[j]],
                                    add=True)
                    if j + 2 < GRP:
                        cps[j % 2] = pltpu.async_copy(
                            h1_hbm.at[gbuf.at[j + 2]], rows[j % 2],
                            sems[j % 2])

            return carry

        lax.fori_loop(0, NG // NTILE + 1, body, 0)
        plsc.subcore_barrier()

        def wrbody(k, carry):
            t = sid + k * NTILE

            @pl.when(t < NPIECE)
            def _():
                p = _piece_start(t)
                _fill_iota(ibuf, p, PIECE)
                pltpu.sync_copy(acc.at[ibuf], rowsA)
                pltpu.sync_copy(rowsA, out.at[pl.ds(c * N + p, PIECE)])

            return carry

        lax.fori_loop(0, NPIECE // NTILE + 1, wrbody, 0)
        plsc.subcore_barrier()


def _sc_agg2(h1flat, src2d, dst2d):
    z128 = jnp.zeros((PIECE, 128), jnp.float32)
    return pl.kernel(
        _sc2_body,
        out_type=jax.ShapeDtypeStruct((NCHUNK * N, 128), jnp.float32),
        mesh=_sc_mesh(),
        scratch_types=[
            pltpu.VMEM_SHARED((N, 128), jnp.float32),
            pltpu.VMEM((GRP, BATCH), jnp.int32),
            pltpu.VMEM((GRP, BATCH), jnp.int32),
            pltpu.VMEM((GRP, BATCH), jnp.int32),
            pltpu.VMEM((BATCH, 128), jnp.float32),
            pltpu.VMEM((BATCH, 128), jnp.float32),
            pltpu.VMEM((PIECE,), jnp.int32),
            pltpu.SemaphoreType.DMA,
            pltpu.SemaphoreType.DMA,
        ],
    )(h1flat, src2d, dst2d, z128)


def kernel(x, edge_index, batch, Wl1, bl1, Wr1, Wl2, bl2, Wr2, Wf1, bf1,
           Wf2, bf2):
    src = edge_index[0]
    dst = edge_index[1]

    aggp, degp = _sc_agg1(x, src, dst)

    h1c, dinv = _mm1(x, aggp, degp, Wl1, bl1, Wr1)

    agg2c = _sc_agg2(h1c.reshape(NCHUNK * N, 128),
                     src.reshape(NB // GRP, GRP, BATCH),
                     dst.reshape(NB // GRP, GRP, BATCH)
                     ).reshape(NCHUNK, N, 128)

    g = _mm2(h1c, agg2c, dinv, Wl2, bl2, Wr2, batch)
    return _mm3(g, Wf1, bf1, Wf2, bf2)
